# baseline (speedup 1.0000x reference)
import jax
import jax.numpy as jnp
from jax import lax
from jax.experimental import pallas as pl
from jax.experimental.pallas import tpu as pltpu
from jax.experimental.pallas import tpu_sc as plsc

NUM_ITEMS = 1000000
SIDE_VOCAB = 100000
N_SIDE = 2
EMB = 64
B = 16384
NCTX = 5

NC = 2
NS = 16
L = 16
NW = NC * NS
BW = B // NW
CH = 64
NCHUNK = BW // CH
NBUF = 2


def _softmax3(w0, w1, w2):
    m = jnp.maximum(w0, jnp.maximum(w1, w2))
    e0 = jnp.exp(w0 - m)
    e1 = jnp.exp(w1 - m)
    e2 = jnp.exp(w2 - m)
    s = e0 + e1 + e2
    return e0 / s, e1 / s, e2 / s


def _body(ci_hbm, csi_hbm, ctx_hbm, ein_hbm, eout_hbm, wt_hbm, side_hbm,
          out_hbm, *scratch):
    per = 12
    slots = [scratch[i * per:(i + 1) * per] for i in range(NBUF)]

    wid = lax.axis_index("s") * NC + lax.axis_index("c")
    base0 = wid * BW

    iota16 = lax.iota(jnp.int32, L)

    def issue(k):
        ii, is0, is1, ic2, icf, ri, rs0, rs1, rc, wr, ob, sem = \
            slots[k % NBUF]
        base = base0 + k * CH
        pltpu.sync_copy(ci_hbm.at[pl.ds(base, CH)], ii)
        pltpu.sync_copy(csi_hbm.at[0].at[pl.ds(base, CH)], is0)
        pltpu.sync_copy(csi_hbm.at[1].at[pl.ds(base, CH)], is1)
        pltpu.sync_copy(ctx_hbm.at[pl.ds(base, CH)], ic2)
        for c in range(NCTX):
            for t in range(CH // L):
                v = plsc.load_gather(
                    ic2, [jnp.full((L,), t * L, jnp.int32) + iota16,
                          jnp.full((L,), c, jnp.int32)])
                icf[pl.ds(c * CH + t * L, L)] = v
        hs = [
            pltpu.make_async_copy(ein_hbm.at[ii], ri, sem),
            pltpu.make_async_copy(side_hbm.at[0].at[is0], rs0, sem),
            pltpu.make_async_copy(side_hbm.at[1].at[is1], rs1, sem),
            pltpu.make_async_copy(wt_hbm.at[ii], wr, sem),
        ]
        for c in range(NCTX):
            hs.append(pltpu.make_async_copy(
                eout_hbm.at[icf.at[pl.ds(c * CH, CH)]],
                rc.at[pl.ds(c * CH, CH)], sem))
        for h in hs:
            h.start()
        return hs

    def compute(k):
        ii, is0, is1, ic2, icf, ri, rs0, rs1, rc, wr, ob, sem = \
            slots[k % NBUF]
        zf = jnp.zeros((L,), jnp.float32)

        def group(g, carry):
            lane = jnp.full((L,), g * L, jnp.int32) + iota16
            w0 = plsc.load_gather(wr, [lane, jnp.full((L,), 0, jnp.int32)])
            w1 = plsc.load_gather(wr, [lane, jnp.full((L,), 1, jnp.int32)])
            w2 = plsc.load_gather(wr, [lane, jnp.full((L,), 2, jnp.int32)])
            p0, p1, p2 = _softmax3(w0, w1, w2)

            def dbody(d, accs):
                dv = jnp.full((L,), d, jnp.int32)
                h = (p0 * plsc.load_gather(ri, [lane, dv])
                     + p1 * plsc.load_gather(rs0, [lane, dv])
                     + p2 * plsc.load_gather(rs1, [lane, dv]))
                return tuple(
                    accs[c] + h * plsc.load_gather(
                        rc, [jnp.full((L,), c * CH, jnp.int32) + lane, dv])
                    for c in range(NCTX))

            accs = lax.fori_loop(0, EMB, dbody, (zf,) * NCTX)
            for c in range(NCTX):
                sig = 1.0 / (1.0 + jnp.exp(-accs[c]))
                plsc.store_scatter(
                    ob, [lane, jnp.full((L,), c, jnp.int32)], sig)
            return carry

        lax.fori_loop(0, CH // L, group, 0)

    pending = issue(0)
    for k in range(NCHUNK):
        nxt = issue(k + 1) if k + 1 < NCHUNK else None
        for h in pending:
            h.wait()
        compute(k)
        ob = slots[k % NBUF][10]
        pltpu.sync_copy(ob, out_hbm.at[pl.ds(base0 + k * CH, CH)])
        pending = nxt


def _scratch_types():
    per_slot = [
        pltpu.VMEM((CH,), jnp.int32),
        pltpu.VMEM((CH,), jnp.int32),
        pltpu.VMEM((CH,), jnp.int32),
        pltpu.VMEM((CH, NCTX), jnp.int32),
        pltpu.VMEM((CH * NCTX,), jnp.int32),
        pltpu.VMEM((CH, EMB), jnp.float32),
        pltpu.VMEM((CH, EMB), jnp.float32),
        pltpu.VMEM((CH, EMB), jnp.float32),
        pltpu.VMEM((CH * NCTX, EMB), jnp.float32),
        pltpu.VMEM((CH, N_SIDE + 1), jnp.float32),
        pltpu.VMEM((CH, NCTX), jnp.float32),
        pltpu.SemaphoreType.DMA,
    ]
    return per_slot * NBUF


@jax.jit
def kernel(central_items, central_side_informations, context_items,
           item_embedding_in, item_embedding_out, weights_table, side_tables):
    ci = central_items.astype(jnp.int32)
    csi = central_side_informations.astype(jnp.int32)
    ctx = context_items.astype(jnp.int32)

    mesh = plsc.VectorSubcoreMesh(
        core_axis_name="c", subcore_axis_name="s",
        num_cores=NC, num_subcores=NS)
    run = pl.kernel(
        _body,
        out_type=jax.ShapeDtypeStruct((B, NCTX), jnp.float32),
        mesh=mesh,
        scratch_types=_scratch_types(),
        compiler_params=pltpu.CompilerParams(
            needs_layout_passes=False, use_tc_tiling_on_sc=False),
    )
    return run(ci, csi, ctx, item_embedding_in, item_embedding_out,
               weights_table, side_tables)


# speedup vs baseline: 3.6034x; 3.6034x over previous
import jax
import jax.numpy as jnp
from jax import lax
from jax.experimental import pallas as pl
from jax.experimental.pallas import tpu as pltpu
from jax.experimental.pallas import tpu_sc as plsc

NUM_ITEMS = 1000000
SIDE_VOCAB = 100000
N_SIDE = 2
EMB = 64
B = 16384
NCTX = 5

NC = 2
NS = 16
L = 16
NW = NC * NS
BW = B // NW
CH = 64
NCHUNK = BW // CH
NBUF = 2


def _softmax3(w0, w1, w2):
    m = jnp.maximum(w0, jnp.maximum(w1, w2))
    e0 = jnp.exp(w0 - m)
    e1 = jnp.exp(w1 - m)
    e2 = jnp.exp(w2 - m)
    s = e0 + e1 + e2
    return e0 / s, e1 / s, e2 / s


def _body(ci_hbm, csi_hbm, ctxT_hbm, ein_hbm, eout_hbm, w0_hbm, w1_hbm,
          w2_hbm, side_hbm, out_hbm, *scratch):
    per = 12
    slots = [scratch[i * per:(i + 1) * per] for i in range(NBUF)]

    wid = lax.axis_index("s") * NC + lax.axis_index("c")
    base0 = wid * BW

    iota16 = lax.iota(jnp.int32, L)

    def issue_idx(k):
        ii, is0, is1, icf, ri, rs0, rs1, rc, w3, ob, semi, sem = \
            slots[k % NBUF]
        base = base0 + k * CH
        hs = [
            pltpu.make_async_copy(ci_hbm.at[pl.ds(base, CH)], ii, semi),
            pltpu.make_async_copy(csi_hbm.at[0].at[pl.ds(base, CH)], is0,
                                  semi),
            pltpu.make_async_copy(csi_hbm.at[1].at[pl.ds(base, CH)], is1,
                                  semi),
        ]
        for c in range(NCTX):
            hs.append(pltpu.make_async_copy(
                ctxT_hbm.at[c].at[pl.ds(base, CH)],
                icf.at[pl.ds(c * CH, CH)], semi))
        for h in hs:
            h.start()
        return hs

    def issue_gather(k, idx_pending):
        ii, is0, is1, icf, ri, rs0, rs1, rc, w3, ob, semi, sem = \
            slots[k % NBUF]
        for h in idx_pending:
            h.wait()
        hs = [
            pltpu.make_async_copy(ein_hbm.at[ii], ri, sem),
            pltpu.make_async_copy(side_hbm.at[0].at[is0], rs0, sem),
            pltpu.make_async_copy(side_hbm.at[1].at[is1], rs1, sem),
            pltpu.make_async_copy(w0_hbm.at[ii], w3.at[pl.ds(0, CH)], sem),
            pltpu.make_async_copy(w1_hbm.at[ii], w3.at[pl.ds(CH, CH)], sem),
            pltpu.make_async_copy(w2_hbm.at[ii], w3.at[pl.ds(2 * CH, CH)],
                                  sem),
        ]
        for c in range(NCTX):
            hs.append(pltpu.make_async_copy(
                eout_hbm.at[icf.at[pl.ds(c * CH, CH)]],
                rc.at[pl.ds(c * CH, CH)], sem))
        for h in hs:
            h.start()
        return hs

    def compute(k):
        ii, is0, is1, icf, ri, rs0, rs1, rc, w3, ob, semi, sem = \
            slots[k % NBUF]
        zf = jnp.zeros((L,), jnp.float32)

        def group(g, carry):
            o = g * L
            lane = jnp.full((L,), o, jnp.int32) + iota16
            p0, p1, p2 = _softmax3(
                w3[pl.ds(o, L)], w3[pl.ds(CH + o, L)],
                w3[pl.ds(2 * CH + o, L)])

            def dbody(d, accs):
                dv = jnp.full((L,), d, jnp.int32)
                h = (p0 * plsc.load_gather(ri, [lane, dv])
                     + p1 * plsc.load_gather(rs0, [lane, dv])
                     + p2 * plsc.load_gather(rs1, [lane, dv]))
                return tuple(
                    accs[c] + h * plsc.load_gather(
                        rc, [jnp.full((L,), c * CH, jnp.int32) + lane, dv])
                    for c in range(NCTX))

            accs = lax.fori_loop(0, EMB, dbody, (zf,) * NCTX)
            for c in range(NCTX):
                sig = 1.0 / (1.0 + jnp.exp(-accs[c]))
                plsc.store_scatter(
                    ob, [lane, jnp.full((L,), c, jnp.int32)], sig)
            return carry

        lax.fori_loop(0, CH // L, group, 0)

    idx_pending = issue_idx(0)
    gather_pending = issue_gather(0, idx_pending)
    idx_pending = issue_idx(1)
    for k in range(NCHUNK):
        if k + 1 < NCHUNK:
            nxt = issue_gather(k + 1, idx_pending)
            if k + 2 < NCHUNK:
                idx_pending = issue_idx(k + 2)
        else:
            nxt = None
        for h in gather_pending:
            h.wait()
        compute(k)
        ob = slots[k % NBUF][9]
        pltpu.sync_copy(ob, out_hbm.at[pl.ds(base0 + k * CH, CH)])
        gather_pending = nxt


def _scratch_types():
    per_slot = [
        pltpu.VMEM((CH,), jnp.int32),
        pltpu.VMEM((CH,), jnp.int32),
        pltpu.VMEM((CH,), jnp.int32),
        pltpu.VMEM((CH * NCTX,), jnp.int32),
        pltpu.VMEM((CH, EMB), jnp.float32),
        pltpu.VMEM((CH, EMB), jnp.float32),
        pltpu.VMEM((CH, EMB), jnp.float32),
        pltpu.VMEM((CH * NCTX, EMB), jnp.float32),
        pltpu.VMEM((CH * 3,), jnp.float32),
        pltpu.VMEM((CH, NCTX), jnp.float32),
        pltpu.SemaphoreType.DMA,
        pltpu.SemaphoreType.DMA,
    ]
    return per_slot * NBUF


@jax.jit
def kernel(central_items, central_side_informations, context_items,
           item_embedding_in, item_embedding_out, weights_table, side_tables):
    ci = central_items.astype(jnp.int32)
    csi = central_side_informations.astype(jnp.int32)
    ctxT = context_items.astype(jnp.int32).T
    w0 = weights_table[:, 0]
    w1 = weights_table[:, 1]
    w2 = weights_table[:, 2]

    mesh = plsc.VectorSubcoreMesh(
        core_axis_name="c", subcore_axis_name="s",
        num_cores=NC, num_subcores=NS)
    run = pl.kernel(
        _body,
        out_type=jax.ShapeDtypeStruct((B, NCTX), jnp.float32),
        mesh=mesh,
        scratch_types=_scratch_types(),
        compiler_params=pltpu.CompilerParams(
            needs_layout_passes=False, use_tc_tiling_on_sc=False),
    )
    return run(ci, csi, ctxT, item_embedding_in, item_embedding_out,
               w0, w1, w2, side_tables)


# speedup vs baseline: 6.3362x; 1.7584x over previous
import jax
import jax.numpy as jnp
from jax import lax
from jax.experimental import pallas as pl
from jax.experimental.pallas import tpu as pltpu
from jax.experimental.pallas import tpu_sc as plsc

NUM_ITEMS = 1000000
SIDE_VOCAB = 100000
N_SIDE = 2
EMB = 64
B = 16384
NCTX = 5

NC = 2
NS = 16
L = 16
NW = NC * NS
BW = B // NW
CH = 32
NCHUNK = BW // CH
NBUF = 2
PEMB = 2 * EMB


def _widen_body(x_ref, o_ref):
    o_ref[:, 0:EMB] = jnp.swapaxes(x_ref[...], 0, 1)


def _widen_table(xT, v, bc):
    return pl.pallas_call(
        _widen_body,
        grid=(pl.cdiv(v, bc),),
        in_specs=[pl.BlockSpec((EMB, bc), lambda c: (0, c))],
        out_specs=pl.BlockSpec((bc, PEMB), lambda c: (c, 0)),
        out_shape=jax.ShapeDtypeStruct((v, PEMB), jnp.float32),
    )(xT)


def _widen_side_body(x_ref, o_ref):
    o_ref[0, :, 0:EMB] = jnp.swapaxes(x_ref[0], 0, 1)


def _widen_side(sT, v, bc):
    return pl.pallas_call(
        _widen_side_body,
        grid=(N_SIDE, pl.cdiv(v, bc)),
        in_specs=[pl.BlockSpec((1, EMB, bc), lambda j, c: (j, 0, c))],
        out_specs=pl.BlockSpec((1, bc, PEMB), lambda j, c: (j, c, 0)),
        out_shape=jax.ShapeDtypeStruct((N_SIDE, v, PEMB), jnp.float32),
    )(sT)


def _softmax3(w0, w1, w2):
    m = jnp.maximum(w0, jnp.maximum(w1, w2))
    e0 = jnp.exp(w0 - m)
    e1 = jnp.exp(w1 - m)
    e2 = jnp.exp(w2 - m)
    s = e0 + e1 + e2
    return e0 / s, e1 / s, e2 / s


def _body(ci_hbm, csi_hbm, ctx_hbm, ein_hbm, eout_hbm, w0_hbm, w1_hbm,
          w2_hbm, side_hbm, out_hbm, *scratch):
    per = 12
    slots = [scratch[i * per:(i + 1) * per] for i in range(NBUF)]

    wid = lax.axis_index("s") * NC + lax.axis_index("c")
    base0 = wid * BW

    iota16 = lax.iota(jnp.int32, L)

    def issue_idx(k):
        ii, is0, is1, icf, ri, rs0, rs1, rc, w3, ob, semi, sem = \
            slots[k % NBUF]
        base = base0 + k * CH
        hs = [
            pltpu.make_async_copy(ci_hbm.at[pl.ds(base, CH)], ii, semi),
            pltpu.make_async_copy(csi_hbm.at[pl.ds(base, CH)], is0, semi),
            pltpu.make_async_copy(csi_hbm.at[pl.ds(B + base, CH)], is1,
                                  semi),
        ]
        for c in range(NCTX):
            hs.append(pltpu.make_async_copy(
                ctx_hbm.at[pl.ds(c * B + base, CH)],
                icf.at[pl.ds(c * CH, CH)], semi))
        for h in hs:
            h.start()
        return hs

    def issue_gather(k, idx_pending):
        ii, is0, is1, icf, ri, rs0, rs1, rc, w3, ob, semi, sem = \
            slots[k % NBUF]
        for h in idx_pending:
            h.wait()
        hs = [
            pltpu.make_async_copy(ein_hbm.at[ii], ri, sem),
            pltpu.make_async_copy(side_hbm.at[0].at[is0], rs0, sem),
            pltpu.make_async_copy(side_hbm.at[1].at[is1], rs1, sem),
            pltpu.make_async_copy(w0_hbm.at[ii], w3.at[pl.ds(0, CH)], sem),
            pltpu.make_async_copy(w1_hbm.at[ii], w3.at[pl.ds(CH, CH)], sem),
            pltpu.make_async_copy(w2_hbm.at[ii], w3.at[pl.ds(2 * CH, CH)],
                                  sem),
        ]
        for c in range(NCTX):
            hs.append(pltpu.make_async_copy(
                eout_hbm.at[icf.at[pl.ds(c * CH, CH)]],
                rc.at[pl.ds(c * CH, CH)], sem))
        for h in hs:
            h.start()
        return hs

    def compute(k):
        ii, is0, is1, icf, ri, rs0, rs1, rc, w3, ob, semi, sem = \
            slots[k % NBUF]
        zf = jnp.zeros((L,), jnp.float32)

        def group(g, carry):
            o = g * L
            lane = jnp.full((L,), o, jnp.int32) + iota16
            p0, p1, p2 = _softmax3(
                w3[pl.ds(o, L)], w3[pl.ds(CH + o, L)],
                w3[pl.ds(2 * CH + o, L)])

            def dbody(d, accs):
                dv = jnp.full((L,), d, jnp.int32)
                h = (p0 * plsc.load_gather(ri, [lane, dv])
                     + p1 * plsc.load_gather(rs0, [lane, dv])
                     + p2 * plsc.load_gather(rs1, [lane, dv]))
                return tuple(
                    accs[c] + h * plsc.load_gather(
                        rc, [jnp.full((L,), c * CH, jnp.int32) + lane, dv])
                    for c in range(NCTX))

            accs = lax.fori_loop(0, EMB, dbody, (zf,) * NCTX)
            for c in range(NCTX):
                sig = 1.0 / (1.0 + jnp.exp(-accs[c]))
                plsc.store_scatter(
                    ob, [jnp.full((L,), c * CH, jnp.int32) + lane], sig)
            return carry

        lax.fori_loop(0, CH // L, group, 0)

    idx_pending = issue_idx(0)
    gather_pending = issue_gather(0, idx_pending)
    idx_pending = issue_idx(1)
    for k in range(NCHUNK):
        if k + 1 < NCHUNK:
            nxt = issue_gather(k + 1, idx_pending)
            if k + 2 < NCHUNK:
                idx_pending = issue_idx(k + 2)
        else:
            nxt = None
        for h in gather_pending:
            h.wait()
        compute(k)
        ob = slots[k % NBUF][9]
        base = base0 + k * CH
        for c in range(NCTX):
            pltpu.sync_copy(ob.at[pl.ds(c * CH, CH)],
                            out_hbm.at[pl.ds(c * B + base, CH)])
        gather_pending = nxt


def _scratch_types():
    per_slot = [
        pltpu.VMEM((CH,), jnp.int32),
        pltpu.VMEM((CH,), jnp.int32),
        pltpu.VMEM((CH,), jnp.int32),
        pltpu.VMEM((CH * NCTX,), jnp.int32),
        pltpu.VMEM((CH, PEMB), jnp.float32),
        pltpu.VMEM((CH, PEMB), jnp.float32),
        pltpu.VMEM((CH, PEMB), jnp.float32),
        pltpu.VMEM((CH * NCTX, PEMB), jnp.float32),
        pltpu.VMEM((CH * 3,), jnp.float32),
        pltpu.VMEM((CH * NCTX,), jnp.float32),
        pltpu.SemaphoreType.DMA,
        pltpu.SemaphoreType.DMA,
    ]
    return per_slot * NBUF


@jax.jit
def kernel(central_items, central_side_informations, context_items,
           item_embedding_in, item_embedding_out, weights_table, side_tables):
    ci = central_items.astype(jnp.int32)
    csi = central_side_informations.astype(jnp.int32).reshape(-1)
    ctxf = context_items.astype(jnp.int32).T.reshape(-1)
    w0 = weights_table[:, 0]
    w1 = weights_table[:, 1]
    w2 = weights_table[:, 2]

    ein2 = _widen_table(item_embedding_in.T, NUM_ITEMS, 16384)
    eout2 = _widen_table(item_embedding_out.T, NUM_ITEMS, 16384)
    side2 = _widen_side(jnp.transpose(side_tables, (0, 2, 1)), SIDE_VOCAB,
                        16384)

    mesh = plsc.VectorSubcoreMesh(
        core_axis_name="c", subcore_axis_name="s",
        num_cores=NC, num_subcores=NS)
    run = pl.kernel(
        _body,
        out_type=jax.ShapeDtypeStruct((NCTX * B,), jnp.float32),
        mesh=mesh,
        scratch_types=_scratch_types(),
        compiler_params=pltpu.CompilerParams(
            needs_layout_passes=False, use_tc_tiling_on_sc=True),
    )
    out = run(ci, csi, ctxf, ein2, eout2, w0, w1, w2, side2)
    return out.reshape(NCTX, B).T


# speedup vs baseline: 6.5246x; 1.0297x over previous
import jax
import jax.numpy as jnp
from jax import lax
from jax.experimental import pallas as pl
from jax.experimental.pallas import tpu as pltpu
from jax.experimental.pallas import tpu_sc as plsc

NUM_ITEMS = 1000000
SIDE_VOCAB = 100000
N_SIDE = 2
EMB = 64
B = 16384
NCTX = 5

NC = 2
NS = 16
L = 16
NW = NC * NS
BW = B // NW
CH = 32
NCHUNK = BW // CH
NBUF = 3
PEMB = 2 * EMB


def _widen_body(x_ref, o_ref):
    o_ref[:, 0:EMB] = jnp.swapaxes(x_ref[...], 0, 1)


def _widen_table(xT, v, bc):
    return pl.pallas_call(
        _widen_body,
        grid=(pl.cdiv(v, bc),),
        in_specs=[pl.BlockSpec((EMB, bc), lambda c: (0, c))],
        out_specs=pl.BlockSpec((bc, PEMB), lambda c: (c, 0)),
        out_shape=jax.ShapeDtypeStruct((v, PEMB), jnp.float32),
    )(xT)


def _widen_side_body(x_ref, o_ref):
    o_ref[0, :, 0:EMB] = jnp.swapaxes(x_ref[0], 0, 1)


def _widen_side(sT, v, bc):
    return pl.pallas_call(
        _widen_side_body,
        grid=(N_SIDE, pl.cdiv(v, bc)),
        in_specs=[pl.BlockSpec((1, EMB, bc), lambda j, c: (j, 0, c))],
        out_specs=pl.BlockSpec((1, bc, PEMB), lambda j, c: (j, c, 0)),
        out_shape=jax.ShapeDtypeStruct((N_SIDE, v, PEMB), jnp.float32),
    )(sT)


def _softmax3(w0, w1, w2):
    m = jnp.maximum(w0, jnp.maximum(w1, w2))
    e0 = jnp.exp(w0 - m)
    e1 = jnp.exp(w1 - m)
    e2 = jnp.exp(w2 - m)
    s = e0 + e1 + e2
    return e0 / s, e1 / s, e2 / s


def _body_a(ci_hbm, csi_hbm, ein_hbm, w0_hbm, w1_hbm, w2_hbm, side_hbm,
            hid_hbm, *scratch):
    per = 9
    slots = [scratch[i * per:(i + 1) * per] for i in range(NBUF)]
    wid = lax.axis_index("s") * NC + lax.axis_index("c")
    base0 = wid * BW
    iota16 = lax.iota(jnp.int32, L)

    def issue_idx(k):
        ii, is0, is1, ri, rs0, rs1, w3, hb, sem = slots[k % NBUF]
        base = base0 + k * CH
        hs = [
            pltpu.make_async_copy(ci_hbm.at[pl.ds(base, CH)], ii, sem),
            pltpu.make_async_copy(csi_hbm.at[pl.ds(base, CH)], is0, sem),
            pltpu.make_async_copy(csi_hbm.at[pl.ds(B + base, CH)], is1,
                                  sem),
        ]
        for h in hs:
            h.start()
        return hs

    def issue_gather(k, idx_pending):
        ii, is0, is1, ri, rs0, rs1, w3, hb, sem = slots[k % NBUF]
        for h in idx_pending:
            h.wait()
        hs = [
            pltpu.make_async_copy(ein_hbm.at[ii], ri, sem),
            pltpu.make_async_copy(side_hbm.at[0].at[is0], rs0, sem),
            pltpu.make_async_copy(side_hbm.at[1].at[is1], rs1, sem),
            pltpu.make_async_copy(w0_hbm.at[ii], w3.at[pl.ds(0, CH)], sem),
            pltpu.make_async_copy(w1_hbm.at[ii], w3.at[pl.ds(CH, CH)], sem),
            pltpu.make_async_copy(w2_hbm.at[ii], w3.at[pl.ds(2 * CH, CH)],
                                  sem),
        ]
        for h in hs:
            h.start()
        return hs

    def compute(k):
        ii, is0, is1, ri, rs0, rs1, w3, hb, sem = slots[k % NBUF]

        def group(g, carry):
            o = g * L
            lane = jnp.full((L,), o, jnp.int32) + iota16
            lane64 = lane * EMB
            p0, p1, p2 = _softmax3(
                w3[pl.ds(o, L)], w3[pl.ds(CH + o, L)],
                w3[pl.ds(2 * CH + o, L)])

            def dbody(d, carry2):
                dv = jnp.full((L,), d, jnp.int32)
                h = (p0 * plsc.load_gather(ri, [lane, dv])
                     + p1 * plsc.load_gather(rs0, [lane, dv])
                     + p2 * plsc.load_gather(rs1, [lane, dv]))
                plsc.store_scatter(hb, [lane64 + dv], h)
                return carry2

            lax.fori_loop(0, EMB, dbody, 0)
            return carry

        lax.fori_loop(0, CH // L, group, 0)

    pend = [issue_gather(0, issue_idx(0))]
    for k in range(1, NBUF - 1):
        pend.append(issue_gather(k, issue_idx(k)))
    for k in range(NCHUNK):
        if k + NBUF - 1 < NCHUNK:
            pend.append(issue_gather(k + NBUF - 1,
                                     issue_idx(k + NBUF - 1)))
        for h in pend.pop(0):
            h.wait()
        compute(k)
        hb = slots[k % NBUF][7]
        pltpu.sync_copy(
            hb, hid_hbm.at[pl.ds((base0 + k * CH) * EMB, CH * EMB)])


def _scratch_a():
    per_slot = [
        pltpu.VMEM((CH,), jnp.int32),
        pltpu.VMEM((CH,), jnp.int32),
        pltpu.VMEM((CH,), jnp.int32),
        pltpu.VMEM((CH, PEMB), jnp.float32),
        pltpu.VMEM((CH, PEMB), jnp.float32),
        pltpu.VMEM((CH, PEMB), jnp.float32),
        pltpu.VMEM((CH * 3,), jnp.float32),
        pltpu.VMEM((CH * EMB,), jnp.float32),
        pltpu.SemaphoreType.DMA,
    ]
    return per_slot * NBUF


def _body_b(ctx_hbm, eout_hbm, hid_hbm, out_hbm, *scratch):
    per = 5
    slots = [scratch[i * per:(i + 1) * per] for i in range(NBUF)]
    wid = lax.axis_index("s") * NC + lax.axis_index("c")
    base0 = wid * BW
    iota16 = lax.iota(jnp.int32, L)

    def issue_idx(k):
        icf, rc, hbv, ob, sem = slots[k % NBUF]
        base = base0 + k * CH
        hs = [pltpu.make_async_copy(
            hid_hbm.at[pl.ds(base * EMB, CH * EMB)], hbv, sem)]
        for c in range(NCTX):
            hs.append(pltpu.make_async_copy(
                ctx_hbm.at[pl.ds(c * B + base, CH)],
                icf.at[pl.ds(c * CH, CH)], sem))
        for h in hs:
            h.start()
        return hs

    def issue_gather(k, idx_pending):
        icf, rc, hbv, ob, sem = slots[k % NBUF]
        for h in idx_pending:
            h.wait()
        hs = []
        for c in range(NCTX):
            hs.append(pltpu.make_async_copy(
                eout_hbm.at[icf.at[pl.ds(c * CH, CH)]],
                rc.at[pl.ds(c * CH, CH)], sem))
        for h in hs:
            h.start()
        return hs

    def compute(k):
        icf, rc, hbv, ob, sem = slots[k % NBUF]
        zf = jnp.zeros((L,), jnp.float32)

        def group(g, carry):
            o = g * L
            lane = jnp.full((L,), o, jnp.int32) + iota16
            lane64 = lane * EMB

            def dbody(d, accs):
                dv = jnp.full((L,), d, jnp.int32)
                hv = plsc.load_gather(hbv, [lane64 + dv])
                return tuple(
                    accs[c] + hv * plsc.load_gather(
                        rc, [jnp.full((L,), c * CH, jnp.int32) + lane, dv])
                    for c in range(NCTX))

            accs = lax.fori_loop(0, EMB, dbody, (zf,) * NCTX)
            for c in range(NCTX):
                sig = 1.0 / (1.0 + jnp.exp(-accs[c]))
                plsc.store_scatter(
                    ob, [jnp.full((L,), c * CH, jnp.int32) + lane], sig)
            return carry

        lax.fori_loop(0, CH // L, group, 0)

    pend = [issue_gather(0, issue_idx(0))]
    for k in range(1, NBUF - 1):
        pend.append(issue_gather(k, issue_idx(k)))
    for k in range(NCHUNK):
        if k + NBUF - 1 < NCHUNK:
            pend.append(issue_gather(k + NBUF - 1,
                                     issue_idx(k + NBUF - 1)))
        for h in pend.pop(0):
            h.wait()
        compute(k)
        ob = slots[k % NBUF][3]
        base = base0 + k * CH
        for c in range(NCTX):
            pltpu.sync_copy(ob.at[pl.ds(c * CH, CH)],
                            out_hbm.at[pl.ds(c * B + base, CH)])


def _scratch_b():
    per_slot = [
        pltpu.VMEM((CH * NCTX,), jnp.int32),
        pltpu.VMEM((CH * NCTX, PEMB), jnp.float32),
        pltpu.VMEM((CH * EMB,), jnp.float32),
        pltpu.VMEM((CH * NCTX,), jnp.float32),
        pltpu.SemaphoreType.DMA,
    ]
    return per_slot * NBUF


_SC_PARAMS = pltpu.CompilerParams(
    needs_layout_passes=False, use_tc_tiling_on_sc=True)


@jax.jit
def kernel(central_items, central_side_informations, context_items,
           item_embedding_in, item_embedding_out, weights_table, side_tables):
    ci = central_items.astype(jnp.int32)
    csi = central_side_informations.astype(jnp.int32).reshape(-1)
    ctxf = context_items.astype(jnp.int32).T.reshape(-1)
    w0 = weights_table[:, 0]
    w1 = weights_table[:, 1]
    w2 = weights_table[:, 2]

    side2 = _widen_side(jnp.transpose(side_tables, (0, 2, 1)), SIDE_VOCAB,
                        16384)
    ein2 = _widen_table(item_embedding_in.T, NUM_ITEMS, 16384)

    mesh = plsc.VectorSubcoreMesh(
        core_axis_name="c", subcore_axis_name="s",
        num_cores=NC, num_subcores=NS)
    run_a = pl.kernel(
        _body_a,
        out_type=jax.ShapeDtypeStruct((B * EMB,), jnp.float32),
        mesh=mesh, scratch_types=_scratch_a(), compiler_params=_SC_PARAMS)
    hid = run_a(ci, csi, ein2, w0, w1, w2, side2)

    eout2 = _widen_table(item_embedding_out.T, NUM_ITEMS, 16384)

    run_b = pl.kernel(
        _body_b,
        out_type=jax.ShapeDtypeStruct((NCTX * B,), jnp.float32),
        mesh=mesh, scratch_types=_scratch_b(), compiler_params=_SC_PARAMS)
    out = run_b(ctxf, eout2, hid)
    return out.reshape(NCTX, B).T
